# baseline (speedup 1.0000x reference)
import functools

import jax
import jax.numpy as jnp
from jax import lax
from jax.experimental import pallas as pl
from jax.experimental.pallas import tpu as pltpu
from jax.experimental.pallas import tpu_sc as plsc

N_NODES = 10000
NP = 10240
E_EDGES = 320000
C = 128
EPAD = 360448
NROWS = EPAD // C
NC1 = EPAD // (16 * C)
NC2 = EPAD // (32 * C)
RPT = NP // 16

CC = 64
K = 4


def _mesh():
    return plsc.VectorSubcoreMesh(core_axis_name="c", subcore_axis_name="s")


def _fill_rows(ref, n_rows, n_cols, val):
    def row(i, _):
        def col(k, _):
            ref[i, pl.ds(k * 16, 16)] = jnp.full((16,), val, jnp.float32)
            return 0
        lax.fori_loop(0, n_cols // 16, col, 0)
        return 0
    lax.fori_loop(0, n_rows, row, 0)


def _zero_acc_rows(zbuf, acc, s, n_rows):
    def body(m, _):
        pltpu.sync_copy(zbuf, acc.at[pl.ds(s * RPT + m * n_rows, n_rows)])
        return 0
    lax.fori_loop(0, RPT // n_rows, body, 0)


def _make_deg_kernel():
    ib = 8
    nblk = NC2 // ib

    @functools.partial(
        pl.kernel,
        out_type=jax.ShapeDtypeStruct((2, NP, 128), jnp.float32),
        mesh=_mesh(),
        scratch_types=[
            pltpu.VMEM_SHARED((NP, 128), jnp.float32),
            pltpu.VMEM((ib, C), jnp.int32),
            pltpu.VMEM((C, 128), jnp.float32),
            pltpu.VMEM((C, 128), jnp.float32),
        ],
    )
    def deg_kernel(dst_hbm, out_hbm, acc, didx, ones_v, zbuf):
        c = lax.axis_index("c")
        s = lax.axis_index("s")
        wid = c * 16 + s
        _fill_rows(ones_v, C, 128, 1.0)
        _fill_rows(zbuf, C, 128, 0.0)
        _zero_acc_rows(zbuf, acc, s, C)
        plsc.subcore_barrier()

        def blk(b, _):
            pltpu.sync_copy(dst_hbm.at[wid, pl.ds(b * ib, ib)], didx)

            def body(j, _):
                pltpu.sync_copy(ones_v, acc.at[didx.at[j]], add=True)
                return 0
            lax.fori_loop(0, ib, body, 0)
            return 0
        lax.fori_loop(0, nblk, blk, 0)
        plsc.subcore_barrier()
        pltpu.sync_copy(acc.at[pl.ds(s * RPT, RPT)],
                        out_hbm.at[c, pl.ds(s * RPT, RPT)])

    return deg_kernel


def _make_msg_kernel(feature_split):
    n_chunks = NC1 if feature_split else NC2
    ib = 16 if feature_split else 8
    nblk = n_chunks // ib

    @functools.partial(
        pl.kernel,
        out_type=jax.ShapeDtypeStruct((2, NP, 128), jnp.float32),
        mesh=_mesh(),
        scratch_types=[
            pltpu.VMEM_SHARED((NP, 128), jnp.float32),
            pltpu.VMEM((2, ib, C), jnp.int32),
            pltpu.VMEM((2, ib, C), jnp.int32),
            pltpu.VMEM((2, C, 128), jnp.float32),
            pltpu.SemaphoreType.DMA((2,)),
        ],
    )
    def msg_kernel(table_hbm, src_hbm, dst_hbm, out_hbm,
                   acc, sidx, didx, msg, gsem):
        c = lax.axis_index("c")
        s = lax.axis_index("s")
        _fill_rows(msg.at[0], C, 128, 0.0)
        _zero_acc_rows(msg.at[0], acc, s, C)
        plsc.subcore_barrier()

        def load_idx(b):
            p = lax.rem(b, 2)
            if feature_split:
                pltpu.sync_copy(src_hbm.at[c, s, pl.ds(b * ib, ib)], sidx.at[p])
                pltpu.sync_copy(dst_hbm.at[s, pl.ds(b * ib, ib)], didx.at[p])
            else:
                wid = c * 16 + s
                pltpu.sync_copy(src_hbm.at[wid, pl.ds(b * ib, ib)], sidx.at[p])
                pltpu.sync_copy(dst_hbm.at[wid, pl.ds(b * ib, ib)], didx.at[p])

        def start_gathers(j, buf):
            b = j // ib
            r = lax.rem(j, ib)
            for h in range(C // CC):
                pltpu.async_copy(
                    table_hbm.at[sidx.at[lax.rem(b, 2), r,
                                         pl.ds(h * CC, CC)]],
                    msg.at[buf, pl.ds(h * CC, CC)], gsem.at[buf])

        load_idx(0)
        start_gathers(0, 0)
        start_gathers(1, 1)

        def body(j, _):
            buf = lax.rem(j, 2)
            pltpu.make_async_copy(
                table_hbm.at[sidx.at[0, 0]], msg.at[buf], gsem.at[buf]).wait()
            b = j // ib
            pltpu.sync_copy(
                msg.at[buf],
                acc.at[didx.at[lax.rem(b, 2), lax.rem(j, ib)]], add=True)

            nxt = j + 2

            @pl.when(nxt < n_chunks)
            def _():
                @pl.when(lax.rem(nxt, ib) == 0)
                def _():
                    load_idx(nxt // ib)
                start_gathers(nxt, buf)
            return 0
        lax.fori_loop(0, n_chunks, body, 0)
        plsc.subcore_barrier()
        pltpu.sync_copy(acc.at[pl.ds(s * RPT, RPT)],
                        out_hbm.at[c, pl.ds(s * RPT, RPT)])

    return msg_kernel


def _dis_from_deg(deg_ref):
    d = deg_ref[0, :, 0:1] + deg_ref[1, :, 0:1]
    return jnp.where(d > 0.0, lax.rsqrt(d), 0.0)


def _scale1_body(x_ref, w_ref, deg_ref, out_ref):
    dis = _dis_from_deg(deg_ref)
    h = jnp.dot(x_ref[...], w_ref[...], preferred_element_type=jnp.float32)
    out_ref[0] = dis * h


def _mid_body(s1_ref, deg_ref, b1_ref, w2_ref, out_ref):
    dis = _dis_from_deg(deg_ref)
    hcat = jnp.concatenate([dis * s1_ref[0], dis * s1_ref[1]], axis=1)
    hact = jnp.maximum(hcat + b1_ref[0][None, :], 0.0)
    h2 = jnp.dot(hact, w2_ref[...], preferred_element_type=jnp.float32)
    out_ref[...] = dis * h2


def _final_body(s2_ref, deg_ref, b2_ref, out_ref):
    dis = _dis_from_deg(deg_ref)
    out_ref[...] = dis * (s2_ref[0] + s2_ref[1]) + b2_ref[0][None, :]


BM = 512


def _tc_scale1(xp, W1, deg2):
    return pl.pallas_call(
        _scale1_body,
        grid=(NP // BM, 2),
        in_specs=[
            pl.BlockSpec((BM, 128), lambda j, c: (j, 0)),
            pl.BlockSpec((128, 128), lambda j, c: (0, c)),
            pl.BlockSpec((2, BM, 128), lambda j, c: (0, j, 0)),
        ],
        out_specs=pl.BlockSpec((1, BM, 128), lambda j, c: (c, j, 0)),
        out_shape=jax.ShapeDtypeStruct((2, NP, 128), jnp.float32),
    )(xp, W1, deg2)


def _tc_mid(s1, deg2, b1, W2):
    return pl.pallas_call(
        _mid_body,
        grid=(NP // BM,),
        in_specs=[
            pl.BlockSpec((2, BM, 128), lambda j: (0, j, 0)),
            pl.BlockSpec((2, BM, 128), lambda j: (0, j, 0)),
            pl.BlockSpec((1, 256), lambda j: (0, 0)),
            pl.BlockSpec((256, 128), lambda j: (0, 0)),
        ],
        out_specs=pl.BlockSpec((BM, 128), lambda j: (j, 0)),
        out_shape=jax.ShapeDtypeStruct((NP, 128), jnp.float32),
    )(s1, deg2, b1.reshape(1, 256), W2)


def _tc_final(s2, deg2, b2):
    bm = 400
    return pl.pallas_call(
        _final_body,
        grid=(N_NODES // bm,),
        in_specs=[
            pl.BlockSpec((2, bm, 128), lambda j: (0, j, 0)),
            pl.BlockSpec((2, bm, 128), lambda j: (0, j, 0)),
            pl.BlockSpec((1, 128), lambda j: (0, 0)),
        ],
        out_specs=pl.BlockSpec((bm, 128), lambda j: (j, 0)),
        out_shape=jax.ShapeDtypeStruct((N_NODES, 128), jnp.float32),
    )(s2, deg2, b2.reshape(1, 128))


_deg_call = _make_deg_kernel()
_msg1_call = _make_msg_kernel(feature_split=True)
_msg2_call = _make_msg_kernel(feature_split=False)


@jax.jit
def kernel(x, edge_index, W1, b1, W2, b2):
    loop = jnp.arange(N_NODES, dtype=jnp.int32)
    pad = jnp.full((EPAD - E_EDGES - N_NODES,), N_NODES, dtype=jnp.int32)
    src = jnp.concatenate([edge_index[0], loop, pad])
    dst = jnp.concatenate([edge_index[1], loop, pad])
    src16 = src.reshape(16, NC1, C)
    dst16 = dst.reshape(16, NC1, C)
    src32 = src.reshape(32, NC2, C)
    dst32 = dst.reshape(32, NC2, C)
    src1 = jnp.stack([src16, src16 + NP])

    xp = jnp.zeros((NP, 128), jnp.float32).at[:N_NODES].set(x)

    deg2 = _deg_call(dst32)
    hs1 = _tc_scale1(xp, W1, deg2)
    s1 = _msg1_call(hs1.reshape(2 * NP, 128), src1, dst16)
    hs2 = _tc_mid(s1, deg2, b1, W2)
    s2 = _msg2_call(hs2, src32, dst32)
    return _tc_final(s2, deg2, b2)


# speedup vs baseline: 3.5428x; 3.5428x over previous
import functools

import jax
import jax.numpy as jnp
from jax import lax
from jax.experimental import pallas as pl
from jax.experimental.pallas import tpu as pltpu
from jax.experimental.pallas import tpu_sc as plsc

N_NODES = 10000
NP = 10240
E_EDGES = 320000
C = 128
EPAD = 327680
NROWS = EPAD // C
NC1 = EPAD // (16 * C)
NC2 = EPAD // (32 * C)
RPT = NP // 16

CC = 64
K = 4


def _mesh():
    return plsc.VectorSubcoreMesh(core_axis_name="c", subcore_axis_name="s")


def _fill_rows(ref, n_rows, n_cols, val):
    def row(i, _):
        def col(k, _):
            ref[i, pl.ds(k * 16, 16)] = jnp.full((16,), val, jnp.float32)
            return 0
        lax.fori_loop(0, n_cols // 16, col, 0)
        return 0
    lax.fori_loop(0, n_rows, row, 0)


def _zero_acc_rows(zbuf, acc, s, n_rows):
    def body(m, _):
        pltpu.sync_copy(zbuf, acc.at[pl.ds(s * RPT + m * n_rows, n_rows)])
        return 0
    lax.fori_loop(0, RPT // n_rows, body, 0)


def _make_deg_kernel():
    ib = 8
    nblk = NC2 // ib

    @functools.partial(
        pl.kernel,
        out_type=jax.ShapeDtypeStruct((2, NP, 128), jnp.float32),
        mesh=_mesh(),
        scratch_types=[
            pltpu.VMEM_SHARED((NP, 128), jnp.float32),
            pltpu.VMEM((ib, C), jnp.int32),
            pltpu.VMEM((C, 128), jnp.float32),
            pltpu.VMEM((C, 128), jnp.float32),
        ],
    )
    def deg_kernel(dst_hbm, out_hbm, acc, didx, ones_v, zbuf):
        c = lax.axis_index("c")
        s = lax.axis_index("s")
        wid = c * 16 + s
        _fill_rows(ones_v, C, 128, 1.0)
        _fill_rows(zbuf, C, 128, 0.0)
        _zero_acc_rows(zbuf, acc, s, C)
        plsc.subcore_barrier()

        def blk(b, _):
            pltpu.sync_copy(dst_hbm.at[wid, pl.ds(b * ib, ib)], didx)

            def body(j, _):
                pltpu.sync_copy(ones_v, acc.at[didx.at[j]], add=True)
                return 0
            lax.fori_loop(0, ib, body, 0)
            return 0
        lax.fori_loop(0, nblk, blk, 0)
        plsc.subcore_barrier()
        pltpu.sync_copy(acc.at[pl.ds(s * RPT, RPT)],
                        out_hbm.at[c, pl.ds(s * RPT, RPT)])

    return deg_kernel


def _make_msg_kernel(feature_split):
    n_chunks = NC1 if feature_split else NC2
    ib = 16 if feature_split else 8
    nblk = n_chunks // ib

    @functools.partial(
        pl.kernel,
        out_type=jax.ShapeDtypeStruct((2, NP, 128), jnp.float32),
        mesh=_mesh(),
        scratch_types=[
            pltpu.VMEM_SHARED((NP, 128), jnp.float32),
            pltpu.VMEM((2, ib, C), jnp.int32),
            pltpu.VMEM((2, ib, C), jnp.int32),
            pltpu.VMEM((2, C, 128), jnp.float32),
            pltpu.SemaphoreType.DMA((2,)),
        ],
    )
    def msg_kernel(table_hbm, src_hbm, dst_hbm, out_hbm,
                   acc, sidx, didx, msg, gsem):
        c = lax.axis_index("c")
        s = lax.axis_index("s")
        _fill_rows(msg.at[0], C, 128, 0.0)
        _zero_acc_rows(msg.at[0], acc, s, C)
        plsc.subcore_barrier()

        def load_idx(b):
            p = lax.rem(b, 2)
            if feature_split:
                pltpu.sync_copy(src_hbm.at[c, s, pl.ds(b * ib, ib)], sidx.at[p])
                pltpu.sync_copy(dst_hbm.at[s, pl.ds(b * ib, ib)], didx.at[p])
            else:
                wid = c * 16 + s
                pltpu.sync_copy(src_hbm.at[wid, pl.ds(b * ib, ib)], sidx.at[p])
                pltpu.sync_copy(dst_hbm.at[wid, pl.ds(b * ib, ib)], didx.at[p])

        def start_gathers(j, buf):
            b = j // ib
            r = lax.rem(j, ib)
            for h in range(C // CC):
                pltpu.async_copy(
                    table_hbm.at[sidx.at[lax.rem(b, 2), r,
                                         pl.ds(h * CC, CC)]],
                    msg.at[buf, pl.ds(h * CC, CC)], gsem.at[buf])

        load_idx(0)
        start_gathers(0, 0)
        start_gathers(1, 1)

        def body(j, _):
            buf = lax.rem(j, 2)
            pltpu.make_async_copy(
                table_hbm.at[sidx.at[0, 0]], msg.at[buf], gsem.at[buf]).wait()
            b = j // ib
            pltpu.sync_copy(
                msg.at[buf],
                acc.at[didx.at[lax.rem(b, 2), lax.rem(j, ib)]], add=True)

            nxt = j + 2

            @pl.when(nxt < n_chunks)
            def _():
                @pl.when(lax.rem(nxt, ib) == 0)
                def _():
                    load_idx(nxt // ib)
                start_gathers(nxt, buf)
            return 0
        lax.fori_loop(0, n_chunks, body, 0)
        plsc.subcore_barrier()
        pltpu.sync_copy(acc.at[pl.ds(s * RPT, RPT)],
                        out_hbm.at[c, pl.ds(s * RPT, RPT)])

    return msg_kernel


def _dis_from_deg(deg_ref):
    d = deg_ref[0, :, 0:1] + deg_ref[1, :, 0:1] + 1.0
    return lax.rsqrt(d)


def _scale1_body(x_ref, w_ref, deg_ref, out_ref):
    dis = _dis_from_deg(deg_ref)
    h = jnp.dot(x_ref[...], w_ref[...], preferred_element_type=jnp.float32)
    out_ref[0] = dis * h


def _mid_body(s1_ref, hs1_ref, deg_ref, b1_ref, w2_ref, out_ref):
    dis = _dis_from_deg(deg_ref)
    hcat = jnp.concatenate([dis * (s1_ref[0] + hs1_ref[0]),
                            dis * (s1_ref[1] + hs1_ref[1])], axis=1)
    hact = jnp.maximum(hcat + b1_ref[0][None, :], 0.0)
    h2 = jnp.dot(hact, w2_ref[...], preferred_element_type=jnp.float32)
    out_ref[...] = dis * h2


def _final_body(s2_ref, hs2_ref, deg_ref, b2_ref, out_ref):
    dis = _dis_from_deg(deg_ref)
    out_ref[...] = (dis * (s2_ref[0] + s2_ref[1] + hs2_ref[...])
                    + b2_ref[0][None, :])


BM = 512


def _tc_scale1(xp, W1, deg2):
    return pl.pallas_call(
        _scale1_body,
        grid=(NP // BM, 2),
        in_specs=[
            pl.BlockSpec((BM, 128), lambda j, c: (j, 0)),
            pl.BlockSpec((128, 128), lambda j, c: (0, c)),
            pl.BlockSpec((2, BM, 128), lambda j, c: (0, j, 0)),
        ],
        out_specs=pl.BlockSpec((1, BM, 128), lambda j, c: (c, j, 0)),
        out_shape=jax.ShapeDtypeStruct((2, NP, 128), jnp.float32),
    )(xp, W1, deg2)


def _tc_mid(s1, hs1, deg2, b1, W2):
    return pl.pallas_call(
        _mid_body,
        grid=(NP // BM,),
        in_specs=[
            pl.BlockSpec((2, BM, 128), lambda j: (0, j, 0)),
            pl.BlockSpec((2, BM, 128), lambda j: (0, j, 0)),
            pl.BlockSpec((2, BM, 128), lambda j: (0, j, 0)),
            pl.BlockSpec((1, 256), lambda j: (0, 0)),
            pl.BlockSpec((256, 128), lambda j: (0, 0)),
        ],
        out_specs=pl.BlockSpec((BM, 128), lambda j: (j, 0)),
        out_shape=jax.ShapeDtypeStruct((NP, 128), jnp.float32),
    )(s1, hs1, deg2, b1.reshape(1, 256), W2)


def _tc_final(s2, hs2, deg2, b2):
    bm = 400
    return pl.pallas_call(
        _final_body,
        grid=(N_NODES // bm,),
        in_specs=[
            pl.BlockSpec((2, bm, 128), lambda j: (0, j, 0)),
            pl.BlockSpec((bm, 128), lambda j: (j, 0)),
            pl.BlockSpec((2, bm, 128), lambda j: (0, j, 0)),
            pl.BlockSpec((1, 128), lambda j: (0, 0)),
        ],
        out_specs=pl.BlockSpec((bm, 128), lambda j: (j, 0)),
        out_shape=jax.ShapeDtypeStruct((N_NODES, 128), jnp.float32),
    )(s2, hs2, deg2, b2.reshape(1, 128))


_deg_call = _make_deg_kernel()
_msg1_call = _make_msg_kernel(feature_split=True)
_msg2_call = _make_msg_kernel(feature_split=False)


@jax.jit
def kernel(x, edge_index, W1, b1, W2, b2):
    pad = jnp.full((EPAD - E_EDGES,), N_NODES, dtype=jnp.int32)
    src = jnp.concatenate([edge_index[0], pad])
    dst = jnp.concatenate([edge_index[1], pad])
    src16 = src.reshape(16, NC1, C)
    dst16 = dst.reshape(16, NC1, C)
    src32 = src.reshape(32, NC2, C)
    dst32 = dst.reshape(32, NC2, C)
    src1 = jnp.stack([src16, src16 + NP])

    xp = jnp.zeros((NP, 128), jnp.float32).at[:N_NODES].set(x)

    deg2 = _deg_call(dst32)
    hs1 = _tc_scale1(xp, W1, deg2)
    s1 = _msg1_call(hs1.reshape(2 * NP, 128), src1, dst16)
    hs2 = _tc_mid(s1, hs1, deg2, b1, W2)
    s2 = _msg2_call(hs2, src32, dst32)
    return _tc_final(s2, hs2, deg2, b2)


# speedup vs baseline: 3.5459x; 1.0009x over previous
import functools

import jax
import jax.numpy as jnp
from jax import lax
from jax.experimental import pallas as pl
from jax.experimental.pallas import tpu as pltpu
from jax.experimental.pallas import tpu_sc as plsc

N_NODES = 10000
NP = 10240
E_EDGES = 320000
C = 128
EPAD = 327680
NROWS = EPAD // C
NC1 = EPAD // (16 * C)
NC2 = EPAD // (32 * C)
RPT = NP // 16

CC = 64
K = 4


def _mesh():
    return plsc.VectorSubcoreMesh(core_axis_name="c", subcore_axis_name="s")


def _fill_rows(ref, n_rows, n_cols, val):
    def row(i, _):
        def col(k, _):
            ref[i, pl.ds(k * 16, 16)] = jnp.full((16,), val, jnp.float32)
            return 0
        lax.fori_loop(0, n_cols // 16, col, 0)
        return 0
    lax.fori_loop(0, n_rows, row, 0)


def _zero_acc_rows(zbuf, acc, s, n_rows):
    def body(m, _):
        pltpu.sync_copy(zbuf, acc.at[pl.ds(s * RPT + m * n_rows, n_rows)])
        return 0
    lax.fori_loop(0, RPT // n_rows, body, 0)


def _make_deg_kernel():
    ib = 8
    nblk = NC2 // ib

    @functools.partial(
        pl.kernel,
        out_type=jax.ShapeDtypeStruct((2, NP, 128), jnp.float32),
        mesh=_mesh(),
        scratch_types=[
            pltpu.VMEM_SHARED((NP, 128), jnp.float32),
            pltpu.VMEM((ib, C), jnp.int32),
            pltpu.VMEM((C, 128), jnp.float32),
            pltpu.VMEM((C, 128), jnp.float32),
        ],
    )
    def deg_kernel(dst_hbm, out_hbm, acc, didx, ones_v, zbuf):
        c = lax.axis_index("c")
        s = lax.axis_index("s")
        wid = c * 16 + s
        _fill_rows(ones_v, C, 128, 1.0)
        _fill_rows(zbuf, C, 128, 0.0)
        _zero_acc_rows(zbuf, acc, s, C)
        plsc.subcore_barrier()

        def blk(b, _):
            pltpu.sync_copy(dst_hbm.at[wid, pl.ds(b * ib, ib)], didx)

            def body(j, _):
                pltpu.sync_copy(ones_v, acc.at[didx.at[j]], add=True)
                return 0
            lax.fori_loop(0, ib, body, 0)
            return 0
        lax.fori_loop(0, nblk, blk, 0)
        plsc.subcore_barrier()
        pltpu.sync_copy(acc.at[pl.ds(s * RPT, RPT)],
                        out_hbm.at[c, pl.ds(s * RPT, RPT)])

    return deg_kernel


def _make_msg_kernel(feature_split):
    n_chunks = NC1 if feature_split else NC2
    ib = 16
    nblk = n_chunks // ib

    @functools.partial(
        pl.kernel,
        out_type=jax.ShapeDtypeStruct((2, NP, 128), jnp.float32),
        mesh=_mesh(),
        scratch_types=[
            pltpu.VMEM_SHARED((NP, 128), jnp.float32),
            pltpu.VMEM((2, ib, C), jnp.int32),
            pltpu.VMEM((2, ib, C), jnp.int32),
            pltpu.VMEM((2, C, 128), jnp.float32),
            pltpu.SemaphoreType.DMA((2,)),
        ],
    )
    def msg_kernel(table_hbm, src_hbm, dst_hbm, out_hbm,
                   acc, sidx, didx, msg, gsem):
        c = lax.axis_index("c")
        s = lax.axis_index("s")
        _fill_rows(msg.at[0], C, 128, 0.0)
        _zero_acc_rows(msg.at[0], acc, s, C)
        plsc.subcore_barrier()

        def load_idx(b):
            p = lax.rem(b, 2)
            if feature_split:
                pltpu.sync_copy(src_hbm.at[c, s, pl.ds(b * ib, ib)], sidx.at[p])
                pltpu.sync_copy(dst_hbm.at[s, pl.ds(b * ib, ib)], didx.at[p])
            else:
                wid = c * 16 + s
                pltpu.sync_copy(src_hbm.at[wid, pl.ds(b * ib, ib)], sidx.at[p])
                pltpu.sync_copy(dst_hbm.at[wid, pl.ds(b * ib, ib)], didx.at[p])

        def start_gathers(j, buf):
            b = j // ib
            r = lax.rem(j, ib)
            pltpu.async_copy(
                table_hbm.at[sidx.at[lax.rem(b, 2), r]],
                msg.at[buf], gsem.at[buf])

        load_idx(0)
        start_gathers(0, 0)
        start_gathers(1, 1)

        def body(j, _):
            buf = lax.rem(j, 2)
            pltpu.make_async_copy(
                table_hbm.at[sidx.at[0, 0]], msg.at[buf], gsem.at[buf]).wait()
            b = j // ib
            pltpu.sync_copy(
                msg.at[buf],
                acc.at[didx.at[lax.rem(b, 2), lax.rem(j, ib)]], add=True)

            nxt = j + 2

            @pl.when(nxt < n_chunks)
            def _():
                @pl.when(lax.rem(nxt, ib) == 0)
                def _():
                    load_idx(nxt // ib)
                start_gathers(nxt, buf)
            return 0
        lax.fori_loop(0, n_chunks, body, 0)
        plsc.subcore_barrier()
        pltpu.sync_copy(acc.at[pl.ds(s * RPT, RPT)],
                        out_hbm.at[c, pl.ds(s * RPT, RPT)])

    return msg_kernel


def _dis_from_deg(deg_ref):
    d = deg_ref[0, :, 0:1] + deg_ref[1, :, 0:1] + 1.0
    return lax.rsqrt(d)


def _scale1_body(x_ref, w_ref, deg_ref, out_ref):
    dis = _dis_from_deg(deg_ref)
    h = jnp.dot(x_ref[...], w_ref[...], preferred_element_type=jnp.float32)
    out_ref[0] = dis * h


def _mid_body(s1_ref, hs1_ref, deg_ref, b1_ref, w2_ref, out_ref):
    dis = _dis_from_deg(deg_ref)
    hcat = jnp.concatenate([dis * (s1_ref[0] + hs1_ref[0]),
                            dis * (s1_ref[1] + hs1_ref[1])], axis=1)
    hact = jnp.maximum(hcat + b1_ref[0][None, :], 0.0)
    h2 = jnp.dot(hact, w2_ref[...], preferred_element_type=jnp.float32)
    out_ref[...] = dis * h2


def _final_body(s2_ref, hs2_ref, deg_ref, b2_ref, out_ref):
    dis = _dis_from_deg(deg_ref)
    out_ref[...] = (dis * (s2_ref[0] + s2_ref[1] + hs2_ref[...])
                    + b2_ref[0][None, :])


BM = 512


def _tc_scale1(xp, W1, deg2):
    return pl.pallas_call(
        _scale1_body,
        grid=(NP // BM, 2),
        in_specs=[
            pl.BlockSpec((BM, 128), lambda j, c: (j, 0)),
            pl.BlockSpec((128, 128), lambda j, c: (0, c)),
            pl.BlockSpec((2, BM, 128), lambda j, c: (0, j, 0)),
        ],
        out_specs=pl.BlockSpec((1, BM, 128), lambda j, c: (c, j, 0)),
        out_shape=jax.ShapeDtypeStruct((2, NP, 128), jnp.float32),
    )(xp, W1, deg2)


def _tc_mid(s1, hs1, deg2, b1, W2):
    return pl.pallas_call(
        _mid_body,
        grid=(NP // BM,),
        in_specs=[
            pl.BlockSpec((2, BM, 128), lambda j: (0, j, 0)),
            pl.BlockSpec((2, BM, 128), lambda j: (0, j, 0)),
            pl.BlockSpec((2, BM, 128), lambda j: (0, j, 0)),
            pl.BlockSpec((1, 256), lambda j: (0, 0)),
            pl.BlockSpec((256, 128), lambda j: (0, 0)),
        ],
        out_specs=pl.BlockSpec((BM, 128), lambda j: (j, 0)),
        out_shape=jax.ShapeDtypeStruct((NP, 128), jnp.float32),
    )(s1, hs1, deg2, b1.reshape(1, 256), W2)


def _tc_final(s2, hs2, deg2, b2):
    bm = 400
    return pl.pallas_call(
        _final_body,
        grid=(N_NODES // bm,),
        in_specs=[
            pl.BlockSpec((2, bm, 128), lambda j: (0, j, 0)),
            pl.BlockSpec((bm, 128), lambda j: (j, 0)),
            pl.BlockSpec((2, bm, 128), lambda j: (0, j, 0)),
            pl.BlockSpec((1, 128), lambda j: (0, 0)),
        ],
        out_specs=pl.BlockSpec((bm, 128), lambda j: (j, 0)),
        out_shape=jax.ShapeDtypeStruct((N_NODES, 128), jnp.float32),
    )(s2, hs2, deg2, b2.reshape(1, 128))


_deg_call = _make_deg_kernel()
_msg1_call = _make_msg_kernel(feature_split=True)
_msg2_call = _make_msg_kernel(feature_split=False)


@jax.jit
def kernel(x, edge_index, W1, b1, W2, b2):
    pad = jnp.full((EPAD - E_EDGES,), N_NODES, dtype=jnp.int32)
    src = jnp.concatenate([edge_index[0], pad])
    dst = jnp.concatenate([edge_index[1], pad])
    src16 = src.reshape(16, NC1, C)
    dst16 = dst.reshape(16, NC1, C)
    src32 = src.reshape(32, NC2, C)
    dst32 = dst.reshape(32, NC2, C)
    src1 = jnp.stack([src16, src16 + NP])

    xp = jnp.zeros((NP, 128), jnp.float32).at[:N_NODES].set(x)

    deg2 = _deg_call(dst32)
    hs1 = _tc_scale1(xp, W1, deg2)
    s1 = _msg1_call(hs1.reshape(2 * NP, 128), src1, dst16)
    hs2 = _tc_mid(s1, hs1, deg2, b1, W2)
    s2 = _msg2_call(hs2, src32, dst32)
    return _tc_final(s2, hs2, deg2, b2)


# speedup vs baseline: 7.6412x; 2.1550x over previous
import functools

import jax
import jax.numpy as jnp
from jax import lax
from jax.experimental import pallas as pl
from jax.experimental.pallas import tpu as pltpu
from jax.experimental.pallas import tpu_sc as plsc

N_NODES = 10000
NP = 10240
E_EDGES = 320000
C = 128
EPAD = 327680
NROWS = EPAD // C
NC1 = EPAD // (16 * C)
NC2 = EPAD // (32 * C)
RPT = NP // 16

CC = 64
K = 4


def _mesh():
    return plsc.VectorSubcoreMesh(core_axis_name="c", subcore_axis_name="s")


def _fill_rows(ref, n_rows, n_cols, val):
    def row(i, _):
        def col(k, _):
            ref[i, pl.ds(k * 16, 16)] = jnp.full((16,), val, jnp.float32)
            return 0
        lax.fori_loop(0, n_cols // 16, col, 0)
        return 0
    lax.fori_loop(0, n_rows, row, 0)


def _zero_acc_rows(zbuf, acc, s, n_rows):
    def body(m, _):
        pltpu.sync_copy(zbuf, acc.at[pl.ds(s * RPT + m * n_rows, n_rows)])
        return 0
    lax.fori_loop(0, RPT // n_rows, body, 0)


def _make_deg_kernel():
    ib = 8
    nblk = NC2 // ib

    @functools.partial(
        pl.kernel,
        out_type=jax.ShapeDtypeStruct((2, NP, 128), jnp.float32),
        mesh=_mesh(),
        scratch_types=[
            pltpu.VMEM_SHARED((NP, 128), jnp.float32),
            pltpu.VMEM((ib, C), jnp.int32),
            pltpu.VMEM((C, 128), jnp.float32),
            pltpu.VMEM((C, 128), jnp.float32),
        ],
    )
    def deg_kernel(dst_hbm, out_hbm, acc, didx, ones_v, zbuf):
        c = lax.axis_index("c")
        s = lax.axis_index("s")
        wid = c * 16 + s
        _fill_rows(ones_v, C, 128, 1.0)
        _fill_rows(zbuf, C, 128, 0.0)
        _zero_acc_rows(zbuf, acc, s, C)
        plsc.subcore_barrier()

        def blk(b, _):
            pltpu.sync_copy(dst_hbm.at[wid, pl.ds(b * ib, ib)], didx)

            def body(j, _):
                pltpu.sync_copy(ones_v, acc.at[didx.at[j]], add=True)
                return 0
            lax.fori_loop(0, ib, body, 0)
            return 0
        lax.fori_loop(0, nblk, blk, 0)
        plsc.subcore_barrier()
        pltpu.sync_copy(acc.at[pl.ds(s * RPT, RPT)],
                        out_hbm.at[c, pl.ds(s * RPT, RPT)])

    return deg_kernel


def _make_msg_kernel(feature_split):
    n_chunks = NC1 if feature_split else NC2
    ib = 16
    nblk = n_chunks // ib

    @functools.partial(
        pl.kernel,
        out_type=jax.ShapeDtypeStruct((2, NP, 128), jnp.float32),
        mesh=_mesh(),
        scratch_types=[
            pltpu.VMEM_SHARED((NP, 128), jnp.float32),
            pltpu.VMEM((2, ib, C), jnp.int32),
            pltpu.VMEM((2, ib, C), jnp.int32),
            pltpu.VMEM((2, C, 128), jnp.float32),
            pltpu.SemaphoreType.DMA((2,)),
        ],
    )
    def msg_kernel(table_hbm, src_hbm, dst_hbm, out_hbm,
                   acc, sidx, didx, msg, gsem):
        c = lax.axis_index("c")
        s = lax.axis_index("s")
        _fill_rows(msg.at[0], C, 128, 0.0)
        _zero_acc_rows(msg.at[0], acc, s, C)
        plsc.subcore_barrier()

        def load_idx(b):
            p = lax.rem(b, 2)
            if feature_split:
                pltpu.sync_copy(src_hbm.at[c, s, pl.ds(b * ib, ib)], sidx.at[p])
                pltpu.sync_copy(dst_hbm.at[s, pl.ds(b * ib, ib)], didx.at[p])
            else:
                wid = c * 16 + s
                pltpu.sync_copy(src_hbm.at[wid, pl.ds(b * ib, ib)], sidx.at[p])
                pltpu.sync_copy(dst_hbm.at[wid, pl.ds(b * ib, ib)], didx.at[p])

        def start_gathers(j, buf):
            b = j // ib
            r = lax.rem(j, ib)
            pltpu.async_copy(
                table_hbm.at[sidx.at[lax.rem(b, 2), r]],
                msg.at[buf], gsem.at[buf])

        load_idx(0)
        start_gathers(0, 0)
        start_gathers(1, 1)

        def body(j, _):
            buf = lax.rem(j, 2)
            pltpu.make_async_copy(
                table_hbm.at[sidx.at[0, 0]], msg.at[buf], gsem.at[buf]).wait()
            b = j // ib
            pltpu.sync_copy(
                msg.at[buf],
                acc.at[didx.at[lax.rem(b, 2), lax.rem(j, ib)]], add=True)

            nxt = j + 2

            @pl.when(nxt < n_chunks)
            def _():
                @pl.when(lax.rem(nxt, ib) == 0)
                def _():
                    load_idx(nxt // ib)
                start_gathers(nxt, buf)
            return 0
        lax.fori_loop(0, n_chunks, body, 0)
        plsc.subcore_barrier()
        pltpu.sync_copy(acc.at[pl.ds(s * RPT, RPT)],
                        out_hbm.at[c, pl.ds(s * RPT, RPT)])

    return msg_kernel


def _dis_from_deg(deg_ref):
    d = deg_ref[0, :, 0:1] + deg_ref[1, :, 0:1] + 1.0
    return lax.rsqrt(d)


def _scale1_body(x_ref, w_ref, deg_ref, out_ref):
    dis = _dis_from_deg(deg_ref)
    h = jnp.dot(x_ref[...], w_ref[...], preferred_element_type=jnp.float32)
    out_ref[0] = dis * h


def _mid_body(s1_ref, hs1_ref, deg_ref, b1_ref, w2_ref, out_ref):
    dis = _dis_from_deg(deg_ref)
    hcat = jnp.concatenate([dis * (s1_ref[0] + hs1_ref[0]),
                            dis * (s1_ref[1] + hs1_ref[1])], axis=1)
    hact = jnp.maximum(hcat + b1_ref[0][None, :], 0.0)
    h2 = jnp.dot(hact, w2_ref[...], preferred_element_type=jnp.float32)
    out_ref[...] = dis * h2


def _final_body(s2_ref, hs2_ref, deg_ref, b2_ref, out_ref):
    dis = _dis_from_deg(deg_ref)
    out_ref[...] = (dis * (s2_ref[0] + s2_ref[1] + hs2_ref[...])
                    + b2_ref[0][None, :])


BM = 512


def _tc_scale1(xp, W1, deg2):
    return pl.pallas_call(
        _scale1_body,
        grid=(NP // BM, 2),
        in_specs=[
            pl.BlockSpec((BM, 128), lambda j, c: (j, 0)),
            pl.BlockSpec((128, 128), lambda j, c: (0, c)),
            pl.BlockSpec((2, BM, 128), lambda j, c: (0, j, 0)),
        ],
        out_specs=pl.BlockSpec((1, BM, 128), lambda j, c: (c, j, 0)),
        out_shape=jax.ShapeDtypeStruct((2, NP, 128), jnp.float32),
    )(xp, W1, deg2)


def _tc_mid(s1, hs1, deg2, b1, W2):
    return pl.pallas_call(
        _mid_body,
        grid=(NP // BM,),
        in_specs=[
            pl.BlockSpec((2, BM, 128), lambda j: (0, j, 0)),
            pl.BlockSpec((2, BM, 128), lambda j: (0, j, 0)),
            pl.BlockSpec((2, BM, 128), lambda j: (0, j, 0)),
            pl.BlockSpec((1, 256), lambda j: (0, 0)),
            pl.BlockSpec((256, 128), lambda j: (0, 0)),
        ],
        out_specs=pl.BlockSpec((BM, 128), lambda j: (j, 0)),
        out_shape=jax.ShapeDtypeStruct((NP, 128), jnp.float32),
    )(s1, hs1, deg2, b1.reshape(1, 256), W2)


def _tc_final(s2, hs2, deg2, b2):
    bm = 400
    return pl.pallas_call(
        _final_body,
        grid=(N_NODES // bm,),
        in_specs=[
            pl.BlockSpec((2, bm, 128), lambda j: (0, j, 0)),
            pl.BlockSpec((bm, 128), lambda j: (j, 0)),
            pl.BlockSpec((2, bm, 128), lambda j: (0, j, 0)),
            pl.BlockSpec((1, 128), lambda j: (0, 0)),
        ],
        out_specs=pl.BlockSpec((bm, 128), lambda j: (j, 0)),
        out_shape=jax.ShapeDtypeStruct((N_NODES, 128), jnp.float32),
    )(s2, hs2, deg2, b2.reshape(1, 128))


_deg_call = _make_deg_kernel()
_msg1_call = _make_msg_kernel(feature_split=True)
_msg2_call = _make_msg_kernel(feature_split=False)


@jax.jit
def kernel(x, edge_index, W1, b1, W2, b2):
    npad = EPAD - E_EDGES
    pad_src = jnp.arange(npad, dtype=jnp.int32) % N_NODES
    pad_dst = N_NODES + jnp.arange(npad, dtype=jnp.int32) % (NP - N_NODES)
    src = jnp.concatenate([edge_index[0], pad_src])
    dst = jnp.concatenate([edge_index[1], pad_dst])
    src16 = src.reshape(16, NC1, C)
    dst16 = dst.reshape(16, NC1, C)
    src32 = src.reshape(32, NC2, C)
    dst32 = dst.reshape(32, NC2, C)
    src1 = jnp.stack([src16, src16 + NP])

    xp = jnp.zeros((NP, 128), jnp.float32).at[:N_NODES].set(x)

    deg2 = _deg_call(dst32)
    hs1 = _tc_scale1(xp, W1, deg2)
    s1 = _msg1_call(hs1.reshape(2 * NP, 128), src1, dst16)
    hs2 = _tc_mid(s1, hs1, deg2, b1, W2)
    s2 = _msg2_call(hs2, src32, dst32)
    return _tc_final(s2, hs2, deg2, b2)


# speedup vs baseline: 7.8489x; 1.0272x over previous
import functools

import jax
import jax.numpy as jnp
from jax import lax
from jax.experimental import pallas as pl
from jax.experimental.pallas import tpu as pltpu
from jax.experimental.pallas import tpu_sc as plsc

N_NODES = 10000
NP = 10240
E_EDGES = 320000
C = 128
EPAD = 327680
NROWS = EPAD // C
NC1 = EPAD // (16 * C)
NC2 = EPAD // (32 * C)
RPT = NP // 16

def _mesh():
    return plsc.VectorSubcoreMesh(core_axis_name="c", subcore_axis_name="s")


def _fill_rows(ref, n_rows, n_cols, val):
    def row(i, _):
        def col(k, _):
            ref[i, pl.ds(k * 16, 16)] = jnp.full((16,), val, jnp.float32)
            return 0
        lax.fori_loop(0, n_cols // 16, col, 0)
        return 0
    lax.fori_loop(0, n_rows, row, 0)


def _zero_acc_rows(zbuf, acc, s, n_rows):
    def body(m, _):
        pltpu.sync_copy(zbuf, acc.at[pl.ds(s * RPT + m * n_rows, n_rows)])
        return 0
    lax.fori_loop(0, RPT // n_rows, body, 0)


def _make_deg_kernel():
    ib = 8
    nblk = NC2 // ib

    @functools.partial(
        pl.kernel,
        out_type=jax.ShapeDtypeStruct((2, NP, 128), jnp.float32),
        mesh=_mesh(),
        scratch_types=[
            pltpu.VMEM_SHARED((NP, 128), jnp.float32),
            pltpu.VMEM((ib, C), jnp.int32),
            pltpu.VMEM((C, 128), jnp.float32),
            pltpu.VMEM((C, 128), jnp.float32),
        ],
    )
    def deg_kernel(dst_hbm, out_hbm, acc, didx, ones_v, zbuf):
        c = lax.axis_index("c")
        s = lax.axis_index("s")
        wid = c * 16 + s
        _fill_rows(ones_v, C, 128, 1.0)
        _fill_rows(zbuf, C, 128, 0.0)
        _zero_acc_rows(zbuf, acc, s, C)
        plsc.subcore_barrier()

        def blk(b, _):
            pltpu.sync_copy(dst_hbm.at[wid, pl.ds(b * ib, ib)], didx)

            def body(j, _):
                pltpu.sync_copy(ones_v, acc.at[didx.at[j]], add=True)
                return 0
            lax.fori_loop(0, ib, body, 0)
            return 0
        lax.fori_loop(0, nblk, blk, 0)
        plsc.subcore_barrier()
        pltpu.sync_copy(acc.at[pl.ds(s * RPT, RPT)],
                        out_hbm.at[c, pl.ds(s * RPT, RPT)])

    return deg_kernel


def _make_msg_kernel(feature_split):
    n_chunks = NC1 if feature_split else NC2
    ib = 16
    nblk = n_chunks // ib

    @functools.partial(
        pl.kernel,
        out_type=jax.ShapeDtypeStruct((2, NP, 128), jnp.float32),
        mesh=_mesh(),
        scratch_types=[
            pltpu.VMEM_SHARED((NP, 128), jnp.float32),
            pltpu.VMEM((2, ib, C), jnp.int32),
            pltpu.VMEM((2, ib, C), jnp.int32),
            pltpu.VMEM((2, C, 128), jnp.float32),
            pltpu.SemaphoreType.DMA((2,)),
        ],
    )
    def msg_kernel(table_hbm, src_hbm, dst_hbm, out_hbm,
                   acc, sidx, didx, msg, gsem):
        c = lax.axis_index("c")
        s = lax.axis_index("s")
        _fill_rows(msg.at[0], C, 128, 0.0)
        _zero_acc_rows(msg.at[0], acc, s, C)
        plsc.subcore_barrier()

        def load_idx(b):
            p = lax.rem(b, 2)
            if feature_split:
                pltpu.sync_copy(src_hbm.at[c, s, pl.ds(b * ib, ib)], sidx.at[p])
                pltpu.sync_copy(dst_hbm.at[s, pl.ds(b * ib, ib)], didx.at[p])
            else:
                wid = c * 16 + s
                pltpu.sync_copy(src_hbm.at[wid, pl.ds(b * ib, ib)], sidx.at[p])
                pltpu.sync_copy(dst_hbm.at[wid, pl.ds(b * ib, ib)], didx.at[p])

        def start_gathers(j, buf):
            b = j // ib
            r = lax.rem(j, ib)
            pltpu.async_copy(
                table_hbm.at[sidx.at[lax.rem(b, 2), r]],
                msg.at[buf], gsem.at[buf])

        load_idx(0)
        start_gathers(0, 0)
        start_gathers(1, 1)

        def body(j, _):
            buf = lax.rem(j, 2)
            pltpu.make_async_copy(
                table_hbm.at[sidx.at[0, 0]], msg.at[buf], gsem.at[buf]).wait()
            b = j // ib
            pltpu.sync_copy(
                msg.at[buf],
                acc.at[didx.at[lax.rem(b, 2), lax.rem(j, ib)]], add=True)

            nxt = j + 2

            @pl.when(nxt < n_chunks)
            def _():
                @pl.when(lax.rem(nxt, ib) == 0)
                def _():
                    load_idx(nxt // ib)
                start_gathers(nxt, buf)
            return 0
        lax.fori_loop(0, n_chunks, body, 0)
        plsc.subcore_barrier()
        pltpu.sync_copy(acc.at[pl.ds(s * RPT, RPT)],
                        out_hbm.at[c, pl.ds(s * RPT, RPT)])

    return msg_kernel


def _dis_from_deg(deg_ref):
    d = deg_ref[0, :, 0:1] + deg_ref[1, :, 0:1] + 1.0
    return lax.rsqrt(d)


def _mm1_body(x_ref, w_ref, out_ref):
    out_ref[...] = jnp.dot(x_ref[...], w_ref[...],
                           preferred_element_type=jnp.float32)


def _scale1_body(h_ref, deg_ref, out_ref):
    dis = _dis_from_deg(deg_ref)
    h = dis * h_ref[...]
    out_ref[0] = h[:, :128]
    out_ref[1] = h[:, 128:]


def _mid_body(s1_ref, hs1_ref, deg_ref, b1_ref, w2_ref, out_ref):
    dis = _dis_from_deg(deg_ref)
    hcat = jnp.concatenate([dis * (s1_ref[0] + hs1_ref[0]),
                            dis * (s1_ref[1] + hs1_ref[1])], axis=1)
    hact = jnp.maximum(hcat + b1_ref[0][None, :], 0.0)
    h2 = jnp.dot(hact, w2_ref[...], preferred_element_type=jnp.float32)
    out_ref[...] = dis * h2


def _final_body(s2_ref, hs2_ref, deg_ref, b2_ref, out_ref):
    dis = _dis_from_deg(deg_ref)
    out_ref[...] = (dis * (s2_ref[0] + s2_ref[1] + hs2_ref[...])
                    + b2_ref[0][None, :])


BM = 512


def _tc_mm1(xp, W1):
    return pl.pallas_call(
        _mm1_body,
        grid=(NP // BM,),
        in_specs=[
            pl.BlockSpec((BM, 128), lambda j: (j, 0)),
            pl.BlockSpec((128, 256), lambda j: (0, 0)),
        ],
        out_specs=pl.BlockSpec((BM, 256), lambda j: (j, 0)),
        out_shape=jax.ShapeDtypeStruct((NP, 256), jnp.float32),
    )(xp, W1)


def _tc_scale1(h1, deg2):
    return pl.pallas_call(
        _scale1_body,
        grid=(NP // BM,),
        in_specs=[
            pl.BlockSpec((BM, 256), lambda j: (j, 0)),
            pl.BlockSpec((2, BM, 128), lambda j: (0, j, 0)),
        ],
        out_specs=pl.BlockSpec((2, BM, 128), lambda j: (0, j, 0)),
        out_shape=jax.ShapeDtypeStruct((2, NP, 128), jnp.float32),
    )(h1, deg2)


def _tc_mid(s1, hs1, deg2, b1, W2):
    return pl.pallas_call(
        _mid_body,
        grid=(NP // BM,),
        in_specs=[
            pl.BlockSpec((2, BM, 128), lambda j: (0, j, 0)),
            pl.BlockSpec((2, BM, 128), lambda j: (0, j, 0)),
            pl.BlockSpec((2, BM, 128), lambda j: (0, j, 0)),
            pl.BlockSpec((1, 256), lambda j: (0, 0)),
            pl.BlockSpec((256, 128), lambda j: (0, 0)),
        ],
        out_specs=pl.BlockSpec((BM, 128), lambda j: (j, 0)),
        out_shape=jax.ShapeDtypeStruct((NP, 128), jnp.float32),
    )(s1, hs1, deg2, b1.reshape(1, 256), W2)


def _tc_final(s2, hs2, deg2, b2):
    bm = 400
    return pl.pallas_call(
        _final_body,
        grid=(N_NODES // bm,),
        in_specs=[
            pl.BlockSpec((2, bm, 128), lambda j: (0, j, 0)),
            pl.BlockSpec((bm, 128), lambda j: (j, 0)),
            pl.BlockSpec((2, bm, 128), lambda j: (0, j, 0)),
            pl.BlockSpec((1, 128), lambda j: (0, 0)),
        ],
        out_specs=pl.BlockSpec((bm, 128), lambda j: (j, 0)),
        out_shape=jax.ShapeDtypeStruct((N_NODES, 128), jnp.float32),
    )(s2, hs2, deg2, b2.reshape(1, 128))


_deg_call = _make_deg_kernel()
_msg1_call = _make_msg_kernel(feature_split=True)
_msg2_call = _make_msg_kernel(feature_split=False)


@jax.jit
def kernel(x, edge_index, W1, b1, W2, b2):
    npad = EPAD - E_EDGES
    pad_src = jnp.arange(npad, dtype=jnp.int32) % N_NODES
    pad_dst = N_NODES + jnp.arange(npad, dtype=jnp.int32) % (NP - N_NODES)
    src = jnp.concatenate([edge_index[0], pad_src])
    dst = jnp.concatenate([edge_index[1], pad_dst])
    src16 = src.reshape(16, NC1, C)
    dst16 = dst.reshape(16, NC1, C)
    src32 = src.reshape(32, NC2, C)
    dst32 = dst.reshape(32, NC2, C)
    src1 = jnp.stack([src16, src16 + NP])

    xp = jnp.zeros((NP, 128), jnp.float32).at[:N_NODES].set(x)

    h1 = _tc_mm1(xp, W1)
    deg2 = _deg_call(dst32)
    hs1 = _tc_scale1(h1, deg2)
    s1 = _msg1_call(hs1.reshape(2 * NP, 128), src1, dst16)
    hs2 = _tc_mid(s1, hs1, deg2, b1, W2)
    s2 = _msg2_call(hs2, src32, dst32)
    return _tc_final(s2, hs2, deg2, b2)
